# baseline (speedup 1.0000x reference)
import functools

import jax
import jax.numpy as jnp
from jax import lax
from jax.experimental import pallas as pl
from jax.experimental.pallas import tpu as pltpu
from jax.experimental.pallas import tpu_sc as plsc

B = 16384
L = 50
L2 = 2 * L
D = 128
DP = D // 2
NW = 32
BPW = B // NW
GRP = 128
VL = 16
NBUF = 4
UNR = 10


def _accum_half(rows, row0, stg, j):
    nch = DP // VL
    def body(l5, accs):
        for u in range(UNR):
            new = []
            for c in range(nch):
                x = rows[row0 + l5 * UNR + u, pl.ds(c * VL, VL)]
                lo = lax.bitcast_convert_type(
                    jnp.left_shift(x, 16), jnp.float32)
                hi = lax.bitcast_convert_type(
                    jnp.bitwise_and(x, -65536), jnp.float32)
                new.append(accs[c] + lo)
                new.append(accs[nch + c] + hi)
            accs = tuple(new[::2] + new[1::2])
        return accs
    zero = jnp.zeros((VL,), jnp.float32)
    accs = lax.fori_loop(0, L // UNR, body, (zero,) * (2 * nch))
    for c in range(2 * nch):
        stg[j, pl.ds(c * VL, VL)] = accs[c]


_sc_mesh = plsc.VectorSubcoreMesh(core_axis_name="c", subcore_axis_name="s")


@functools.partial(
    pl.kernel,
    out_type=(jax.ShapeDtypeStruct((B, D), jnp.float32),
              jax.ShapeDtypeStruct((B, D), jnp.float32)),
    mesh=_sc_mesh,
    scratch_types=[
        pltpu.VMEM((GRP, L2), jnp.int32),
        pltpu.VMEM((NBUF, L2, DP), jnp.int32),
        pltpu.VMEM((GRP, D), jnp.float32),
        pltpu.VMEM((GRP, D), jnp.float32),
        [pltpu.SemaphoreType.DMA] * NBUF,
    ],
    compiler_params=pltpu.CompilerParams(use_tc_tiling_on_sc=False),
)
def _sc_pool(xc_hbm, table_hbm, outw_hbm, outb_hbm, idx, rows, stgw, stgb,
             sems):
    wid = lax.axis_index("s") * 2 + lax.axis_index("c")
    base = wid * BPW

    def issue(r, u):
        pltpu.async_copy(table_hbm.at[idx.at[r]], rows.at[u], sems[u])

    def wait_and_acc(r, u):
        dummy = table_hbm.at[idx.at[r]]
        pltpu.make_async_copy(dummy, rows.at[u], sems[u]).wait()
        _accum_half(rows.at[u], 0, stgw, r)
        _accum_half(rows.at[u], L, stgb, r)

    def group_body(g, _):
        pltpu.sync_copy(xc_hbm.at[pl.ds(base + g * GRP, GRP)], idx)
        for a in range(NBUF - 1):
            issue(a, a)
        def q_body(q, _):
            r = q * NBUF
            for u in range(NBUF):
                issue(r + u + NBUF - 1, (u + NBUF - 1) % NBUF)
                wait_and_acc(r + u, u)
            return 0
        lax.fori_loop(0, GRP // NBUF - 1, q_body, 0)
        issue(GRP - 1, (GRP - 1) % NBUF)
        for u in range(NBUF):
            wait_and_acc(GRP - NBUF + u, u)
        pltpu.sync_copy(stgw, outw_hbm.at[pl.ds(base + g * GRP, GRP)])
        pltpu.sync_copy(stgb, outb_hbm.at[pl.ds(base + g * GRP, GRP)])
        return 0

    lax.fori_loop(0, BPW // GRP, group_body, 0)


def _mlp_body(x1_ref, x2_ref, w2a_ref, w2b_ref, b2_ref, w3_ref, b3_ref,
              w4_ref, b4_ref, o_ref):
    x1 = jnp.maximum(x1_ref[:], 0.0)
    x2 = jnp.maximum(x2_ref[:], 0.0)
    h = (jnp.dot(x1, w2a_ref[:], preferred_element_type=jnp.float32)
         + jnp.dot(x2, w2b_ref[:], preferred_element_type=jnp.float32)
         + b2_ref[:])
    h = jnp.maximum(h, 0.0)
    h = jnp.dot(h, w3_ref[:], preferred_element_type=jnp.float32) + b3_ref[:]
    h = jnp.maximum(h, 0.0)
    o_ref[:] = jnp.dot(h, w4_ref[:], preferred_element_type=jnp.float32) + b4_ref[:]


def _mlp(pw, pb, W2a, W2b, b2, W3, b3, W4p, b4p):
    blk = 512
    return pl.pallas_call(
        _mlp_body,
        grid=(B // blk,),
        in_specs=[
            pl.BlockSpec((blk, D), lambda i: (i, 0)),
            pl.BlockSpec((blk, D), lambda i: (i, 0)),
            pl.BlockSpec((D, 32), lambda i: (0, 0)),
            pl.BlockSpec((D, 32), lambda i: (0, 0)),
            pl.BlockSpec((1, 32), lambda i: (0, 0)),
            pl.BlockSpec((32, 32), lambda i: (0, 0)),
            pl.BlockSpec((1, 32), lambda i: (0, 0)),
            pl.BlockSpec((32, 128), lambda i: (0, 0)),
            pl.BlockSpec((1, 128), lambda i: (0, 0)),
        ],
        out_specs=pl.BlockSpec((blk, 128), lambda i: (i, 0)),
        out_shape=jax.ShapeDtypeStruct((B, 128), jnp.float32),
    )(pw, pb, W2a, W2b, b2, W3, b3, W4p, b4p)


def _pack_table(table):
    ti = lax.bitcast_convert_type(table, jnp.uint32)
    rnd = jnp.bitwise_and(jnp.right_shift(ti, 16), 1) + jnp.uint32(0x7FFF)
    tb = jnp.right_shift(ti + rnd, 16)
    packed = tb[:, :DP] | jnp.left_shift(tb[:, DP:], 16)
    return lax.bitcast_convert_type(packed, jnp.int32)


def kernel(x_w, x_b, table, W2, b2, W3, b3, W4, b4):
    tpk = _pack_table(table)
    xc = jnp.concatenate([x_w.astype(jnp.int32), x_b.astype(jnp.int32)],
                         axis=1)
    pw, pb = _sc_pool(xc, tpk)
    W4p = jnp.pad(W4, ((0, 0), (0, 127)))
    b4p = jnp.pad(b4.reshape(1, 1), ((0, 0), (0, 127)))
    out = _mlp(pw, pb, W2[:D], W2[D:], b2.reshape(1, 32), W3,
               b3.reshape(1, 32), W4p, b4p)
    return out[:, :1]


# speedup vs baseline: 1.0101x; 1.0101x over previous
import functools

import jax
import jax.numpy as jnp
from jax import lax
from jax.experimental import pallas as pl
from jax.experimental.pallas import tpu as pltpu
from jax.experimental.pallas import tpu_sc as plsc

B = 16384
L = 50
L2 = 2 * L
D = 128
DP = D // 2
NW = 32
BPW = B // NW
GRP = 128
VL = 16
NBUF = 4
UNR = 5


def _accum_half(rows, row0, stg, j):
    nch = DP // VL
    def body(l5, accs):
        for u in range(UNR):
            new = []
            for c in range(nch):
                x = rows[row0 + l5 * UNR + u, pl.ds(c * VL, VL)]
                lo = lax.bitcast_convert_type(
                    jnp.left_shift(x, 16), jnp.float32)
                hi = lax.bitcast_convert_type(
                    jnp.bitwise_and(x, -65536), jnp.float32)
                new.append(accs[c] + lo)
                new.append(accs[nch + c] + hi)
            accs = tuple(new[::2] + new[1::2])
        return accs
    zero = jnp.zeros((VL,), jnp.float32)
    accs = lax.fori_loop(0, L // UNR, body, (zero,) * (2 * nch))
    for c in range(2 * nch):
        stg[j, pl.ds(c * VL, VL)] = accs[c]


_sc_mesh = plsc.VectorSubcoreMesh(core_axis_name="c", subcore_axis_name="s")


@functools.partial(
    pl.kernel,
    out_type=(jax.ShapeDtypeStruct((B, D), jnp.float32),
              jax.ShapeDtypeStruct((B, D), jnp.float32)),
    mesh=_sc_mesh,
    scratch_types=[
        pltpu.VMEM((GRP, L2), jnp.int32),
        pltpu.VMEM((NBUF, L2, DP), jnp.int32),
        pltpu.VMEM((GRP, D), jnp.float32),
        pltpu.VMEM((GRP, D), jnp.float32),
        [pltpu.SemaphoreType.DMA] * NBUF,
    ],
    compiler_params=pltpu.CompilerParams(use_tc_tiling_on_sc=False),
)
def _sc_pool(xc_hbm, table_hbm, outw_hbm, outb_hbm, idx, rows, stgw, stgb,
             sems):
    wid = lax.axis_index("s") * 2 + lax.axis_index("c")
    base = wid * BPW

    def issue(r, u):
        pltpu.async_copy(table_hbm.at[idx.at[r]], rows.at[u], sems[u])

    def wait_and_acc(r, u):
        dummy = table_hbm.at[idx.at[r]]
        pltpu.make_async_copy(dummy, rows.at[u], sems[u]).wait()
        _accum_half(rows.at[u], 0, stgw, r)
        _accum_half(rows.at[u], L, stgb, r)

    def group_body(g, _):
        pltpu.sync_copy(xc_hbm.at[pl.ds(base + g * GRP, GRP)], idx)
        for a in range(NBUF - 1):
            issue(a, a)
        def q_body(q, _):
            r = q * NBUF
            for u in range(NBUF):
                issue(r + u + NBUF - 1, (u + NBUF - 1) % NBUF)
                wait_and_acc(r + u, u)
            return 0
        lax.fori_loop(0, GRP // NBUF - 1, q_body, 0)
        issue(GRP - 1, (GRP - 1) % NBUF)
        for u in range(NBUF):
            wait_and_acc(GRP - NBUF + u, u)
        pltpu.sync_copy(stgw, outw_hbm.at[pl.ds(base + g * GRP, GRP)])
        pltpu.sync_copy(stgb, outb_hbm.at[pl.ds(base + g * GRP, GRP)])
        return 0

    lax.fori_loop(0, BPW // GRP, group_body, 0)


def _mlp_body(x1_ref, x2_ref, w2a_ref, w2b_ref, b2_ref, w3_ref, b3_ref,
              w4_ref, b4_ref, o_ref):
    x1 = jnp.maximum(x1_ref[:], 0.0)
    x2 = jnp.maximum(x2_ref[:], 0.0)
    h = (jnp.dot(x1, w2a_ref[:], preferred_element_type=jnp.float32)
         + jnp.dot(x2, w2b_ref[:], preferred_element_type=jnp.float32)
         + b2_ref[:])
    h = jnp.maximum(h, 0.0)
    h = jnp.dot(h, w3_ref[:], preferred_element_type=jnp.float32) + b3_ref[:]
    h = jnp.maximum(h, 0.0)
    o_ref[:] = jnp.dot(h, w4_ref[:], preferred_element_type=jnp.float32) + b4_ref[:]


def _mlp(pw, pb, W2a, W2b, b2, W3, b3, W4p, b4p):
    blk = 512
    return pl.pallas_call(
        _mlp_body,
        grid=(B // blk,),
        in_specs=[
            pl.BlockSpec((blk, D), lambda i: (i, 0)),
            pl.BlockSpec((blk, D), lambda i: (i, 0)),
            pl.BlockSpec((D, 32), lambda i: (0, 0)),
            pl.BlockSpec((D, 32), lambda i: (0, 0)),
            pl.BlockSpec((1, 32), lambda i: (0, 0)),
            pl.BlockSpec((32, 32), lambda i: (0, 0)),
            pl.BlockSpec((1, 32), lambda i: (0, 0)),
            pl.BlockSpec((32, 128), lambda i: (0, 0)),
            pl.BlockSpec((1, 128), lambda i: (0, 0)),
        ],
        out_specs=pl.BlockSpec((blk, 128), lambda i: (i, 0)),
        out_shape=jax.ShapeDtypeStruct((B, 128), jnp.float32),
    )(pw, pb, W2a, W2b, b2, W3, b3, W4p, b4p)


def _pack_table(table):
    ti = lax.bitcast_convert_type(table, jnp.uint32)
    rnd = jnp.bitwise_and(jnp.right_shift(ti, 16), 1) + jnp.uint32(0x7FFF)
    tb = jnp.right_shift(ti + rnd, 16)
    packed = tb[:, :DP] | jnp.left_shift(tb[:, DP:], 16)
    return lax.bitcast_convert_type(packed, jnp.int32)


def kernel(x_w, x_b, table, W2, b2, W3, b3, W4, b4):
    tpk = _pack_table(table)
    xc = jnp.concatenate([x_w.astype(jnp.int32), x_b.astype(jnp.int32)],
                         axis=1)
    pw, pb = _sc_pool(xc, tpk)
    W4p = jnp.pad(W4, ((0, 0), (0, 127)))
    b4p = jnp.pad(b4.reshape(1, 1), ((0, 0), (0, 127)))
    out = _mlp(pw, pb, W2[:D], W2[D:], b2.reshape(1, 32), W3,
               b3.reshape(1, 32), W4p, b4p)
    return out[:, :1]
